# baseline (speedup 1.0000x reference)
import functools
import math

import numpy as np
import jax
import jax.numpy as jnp
from jax import lax
from jax.experimental import pallas as pl
from jax.experimental.pallas import tpu as pltpu
from jax.experimental.pallas import tpu_sc as plsc

D_MODEL = 256
BATCH = 1024
SEQ = 50
B = SEQ * BATCH
NC, NS = 2, 16
NW = NC * NS
ROWS_PER_W = B // NW
CHUNK = 64
NCHUNKS = ROWS_PER_W // CHUNK
LANES = 16
NVEC = D_MODEL // LANES
SCALE = 16.0


def _pe_rows():
    position = np.arange(SEQ, dtype=np.float32)[:, None]
    div_term = np.exp(
        np.arange(0, D_MODEL, 2, dtype=np.float32) * -(math.log(10000.0) / D_MODEL)
    )
    pe = np.zeros((SEQ, D_MODEL), dtype=np.float32)
    pe[:, 0::2] = np.sin(position * div_term)
    pe[:, 1::2] = np.cos(position * div_term)
    return pe


_PE = _pe_rows()


NBUF = 3
GCHUNK = 128
_CHUNKS = []
_off = 0
while _off < ROWS_PER_W:
    n = min(GCHUNK, ROWS_PER_W - _off)
    _CHUNKS.append((_off, n))
    _off += n
NCH = len(_CHUNKS)


def _sc_embed(table, idx_flat, pe):
    mesh = plsc.VectorSubcoreMesh(core_axis_name="c", subcore_axis_name="s")

    @functools.partial(
        pl.kernel,
        mesh=mesh,
        out_type=jax.ShapeDtypeStruct((B, D_MODEL), jnp.float32),
        scratch_types=[
            pltpu.VMEM((ROWS_PER_W,), jnp.int32),
            *[pltpu.VMEM((GCHUNK, D_MODEL), jnp.float32) for _ in range(NBUF)],
            pltpu.VMEM((SEQ, D_MODEL), jnp.float32),
            *[pltpu.SemaphoreType.DMA for _ in range(2 * NBUF)],
        ],
    )
    def k(table_hbm, idx_hbm, pe_hbm, out_hbm, idx_v, *rest):
        bufs = list(rest[:NBUF])
        pe_v = rest[NBUF]
        gsems = list(rest[NBUF + 1 : NBUF + 1 + NBUF])
        ssems = list(rest[NBUF + 1 + NBUF :])

        wid = lax.axis_index("s") * NC + lax.axis_index("c")
        base = pl.multiple_of(wid * ROWS_PER_W, ROWS_PER_W)
        pltpu.sync_copy(pe_hbm, pe_v)
        pltpu.sync_copy(idx_hbm.at[pl.ds(base, ROWS_PER_W)], idx_v)

        def start_gather(ci):
            loff, n = _CHUNKS[ci]
            buf = bufs[ci % NBUF]
            dst = buf if n == GCHUNK else buf.at[pl.ds(0, n)]
            return pltpu.async_copy(
                table_hbm.at[idx_v.at[pl.ds(loff, n)]], dst, gsems[ci % NBUF]
            )

        def compute(ci):
            loff, n = _CHUNKS[ci]
            buf = bufs[ci % NBUF]
            for soff in range(0, n, CHUNK):
                t = (base + loff + soff) // BATCH
                pe_vecs = [pe_v[t, pl.ds(j * LANES, LANES)] for j in range(NVEC)]

                def row_body(r, _):
                    for j in range(NVEC):
                        sl = pl.ds(j * LANES, LANES)
                        buf[r, sl] = buf[r, sl] * SCALE + pe_vecs[j]
                    return 0

                lax.fori_loop(soff, soff + CHUNK, row_body, 0)

        gcp, scp, waited = {}, {}, set()
        for ci in range(min(2, NCH)):
            gcp[ci] = start_gather(ci)
        for ci in range(NCH):
            bi = ci % NBUF
            loff, n = _CHUNKS[ci]
            nxt = ci + 2
            if nxt < NCH:
                prev = nxt - NBUF
                if prev >= 0:
                    scp[prev].wait()
                    waited.add(prev)
                gcp[nxt] = start_gather(nxt)
            gcp[ci].wait()
            compute(ci)
            src = bufs[bi] if n == GCHUNK else bufs[bi].at[pl.ds(0, n)]
            scp[ci] = pltpu.async_copy(
                src,
                out_hbm.at[pl.ds(pl.multiple_of(base + loff, CHUNK), n)],
                ssems[bi],
            )
        for ci in range(NCH):
            if ci not in waited:
                scp[ci].wait()

    return k(table, idx_flat, pe)


def _masks(tgt32):
    def body(tgt_ref, pad_ref, tri_ref):
        pad_ref[...] = tgt_ref[...] == 0
        r = lax.broadcasted_iota(jnp.int32, (SEQ, SEQ), 0)
        c = lax.broadcasted_iota(jnp.int32, (SEQ, SEQ), 1)
        tri_ref[...] = jnp.where(c <= r, 0.0, -jnp.inf).astype(jnp.float32)

    return pl.pallas_call(
        body,
        out_shape=(
            jax.ShapeDtypeStruct((BATCH, SEQ), jnp.bool_),
            jax.ShapeDtypeStruct((SEQ, SEQ), jnp.float32),
        ),
    )(tgt32)


def kernel(tgt, table):
    tgt32 = tgt.astype(jnp.int32)
    idx_flat = jnp.transpose(tgt32).reshape(B)
    emb_flat = _sc_embed(table, idx_flat, jnp.asarray(_PE))
    pad, tri = _masks(tgt32)
    return emb_flat.reshape(SEQ, BATCH, D_MODEL), tri, pad


# speedup vs baseline: 1.0259x; 1.0259x over previous
import functools
import math

import numpy as np
import jax
import jax.numpy as jnp
from jax import lax
from jax.experimental import pallas as pl
from jax.experimental.pallas import tpu as pltpu
from jax.experimental.pallas import tpu_sc as plsc

D_MODEL = 256
BATCH = 1024
SEQ = 50
B = SEQ * BATCH
NC, NS = 2, 16
NW = NC * NS
ROWS_PER_W = B // NW
CHUNK = 64
NCHUNKS = ROWS_PER_W // CHUNK
LANES = 16
NVEC = D_MODEL // LANES
SCALE = 16.0


def _pe_rows():
    position = np.arange(SEQ, dtype=np.float32)[:, None]
    div_term = np.exp(
        np.arange(0, D_MODEL, 2, dtype=np.float32) * -(math.log(10000.0) / D_MODEL)
    )
    pe = np.zeros((SEQ, D_MODEL), dtype=np.float32)
    pe[:, 0::2] = np.sin(position * div_term)
    pe[:, 1::2] = np.cos(position * div_term)
    return pe


_PE = _pe_rows()


NBUF = 6
LOOKAHEAD = 4
GCHUNK = 64
_CHUNKS = []
_off = 0
while _off < ROWS_PER_W:
    n = min(GCHUNK, ROWS_PER_W - _off)
    _CHUNKS.append((_off, n))
    _off += n
NCH = len(_CHUNKS)


def _sc_embed(table, idx_flat, pe):
    mesh = plsc.VectorSubcoreMesh(core_axis_name="c", subcore_axis_name="s")

    @functools.partial(
        pl.kernel,
        mesh=mesh,
        out_type=jax.ShapeDtypeStruct((B, D_MODEL), jnp.float32),
        scratch_types=[
            pltpu.VMEM((ROWS_PER_W,), jnp.int32),
            *[pltpu.VMEM((GCHUNK, D_MODEL), jnp.float32) for _ in range(NBUF)],
            pltpu.VMEM((SEQ, D_MODEL), jnp.float32),
            *[pltpu.SemaphoreType.DMA for _ in range(2 * NBUF)],
        ],
    )
    def k(table_hbm, idx_hbm, pe_hbm, out_hbm, idx_v, *rest):
        bufs = list(rest[:NBUF])
        pe_v = rest[NBUF]
        gsems = list(rest[NBUF + 1 : NBUF + 1 + NBUF])
        ssems = list(rest[NBUF + 1 + NBUF :])

        wid = lax.axis_index("s") * NC + lax.axis_index("c")
        base = pl.multiple_of(wid * ROWS_PER_W, ROWS_PER_W)
        pltpu.sync_copy(pe_hbm, pe_v)
        pltpu.sync_copy(idx_hbm.at[pl.ds(base, ROWS_PER_W)], idx_v)

        def start_gather(ci):
            loff, n = _CHUNKS[ci]
            buf = bufs[ci % NBUF]
            dst = buf if n == GCHUNK else buf.at[pl.ds(0, n)]
            return pltpu.async_copy(
                table_hbm.at[idx_v.at[pl.ds(loff, n)]], dst, gsems[ci % NBUF]
            )

        def compute(ci):
            loff, n = _CHUNKS[ci]
            buf = bufs[ci % NBUF]
            for soff in range(0, n, CHUNK):
                t = (base + loff + soff) // BATCH
                pe_vecs = [pe_v[t, pl.ds(j * LANES, LANES)] for j in range(NVEC)]

                def row_body(r, _):
                    for j in range(NVEC):
                        sl = pl.ds(j * LANES, LANES)
                        buf[r, sl] = buf[r, sl] * SCALE + pe_vecs[j]
                    return 0

                lax.fori_loop(soff, soff + CHUNK, row_body, 0)

        gcp, scp, waited = {}, {}, set()
        for ci in range(min(LOOKAHEAD, NCH)):
            gcp[ci] = start_gather(ci)
        for ci in range(NCH):
            bi = ci % NBUF
            loff, n = _CHUNKS[ci]
            nxt = ci + LOOKAHEAD
            if nxt < NCH:
                prev = nxt - NBUF
                if prev >= 0 and prev in scp:
                    scp[prev].wait()
                    waited.add(prev)
                gcp[nxt] = start_gather(nxt)
            gcp[ci].wait()
            compute(ci)
            src = bufs[bi] if n == GCHUNK else bufs[bi].at[pl.ds(0, n)]
            scp[ci] = pltpu.async_copy(
                src,
                out_hbm.at[pl.ds(pl.multiple_of(base + loff, CHUNK), n)],
                ssems[bi],
            )
        for ci in range(NCH):
            if ci in scp and ci not in waited:
                scp[ci].wait()

    return k(table, idx_flat, pe)


def _masks(tgt32):
    def body(tgt_ref, pad_ref, tri_ref):
        pad_ref[...] = tgt_ref[...] == 0
        r = lax.broadcasted_iota(jnp.int32, (SEQ, SEQ), 0)
        c = lax.broadcasted_iota(jnp.int32, (SEQ, SEQ), 1)
        tri_ref[...] = jnp.where(c <= r, 0.0, -jnp.inf).astype(jnp.float32)

    return pl.pallas_call(
        body,
        out_shape=(
            jax.ShapeDtypeStruct((BATCH, SEQ), jnp.bool_),
            jax.ShapeDtypeStruct((SEQ, SEQ), jnp.float32),
        ),
    )(tgt32)


def kernel(tgt, table):
    tgt32 = tgt.astype(jnp.int32)
    idx_flat = jnp.transpose(tgt32).reshape(B)
    emb_flat = _sc_embed(table, idx_flat, jnp.asarray(_PE))
    pad, tri = _masks(tgt32)
    return emb_flat.reshape(SEQ, BATCH, D_MODEL), tri, pad


# speedup vs baseline: 1.0788x; 1.0516x over previous
import functools
import math

import numpy as np
import jax
import jax.numpy as jnp
from jax import lax
from jax.experimental import pallas as pl
from jax.experimental.pallas import tpu as pltpu
from jax.experimental.pallas import tpu_sc as plsc

D_MODEL = 256
BATCH = 1024
SEQ = 50
B = SEQ * BATCH
NC, NS = 2, 16
NW = NC * NS
ROWS_PER_W = B // NW
CHUNK = 64
NCHUNKS = ROWS_PER_W // CHUNK
LANES = 16
NVEC = D_MODEL // LANES
SCALE = 16.0


def _pe_rows():
    position = np.arange(SEQ, dtype=np.float32)[:, None]
    div_term = np.exp(
        np.arange(0, D_MODEL, 2, dtype=np.float32) * -(math.log(10000.0) / D_MODEL)
    )
    pe = np.zeros((SEQ, D_MODEL), dtype=np.float32)
    pe[:, 0::2] = np.sin(position * div_term)
    pe[:, 1::2] = np.cos(position * div_term)
    return pe


_PE = _pe_rows()


NBUF = 5
LAG = 2
NGROUPS = NCHUNKS // NBUF


def _sc_embed(table, idx_flat, pe):
    mesh = plsc.VectorSubcoreMesh(core_axis_name="c", subcore_axis_name="s")

    @functools.partial(
        pl.kernel,
        mesh=mesh,
        out_type=jax.ShapeDtypeStruct((B, D_MODEL), jnp.float32),
        scratch_types=[
            pltpu.VMEM((ROWS_PER_W,), jnp.int32),
            *[pltpu.VMEM((CHUNK, D_MODEL), jnp.float32) for _ in range(NBUF)],
            pltpu.VMEM((SEQ, D_MODEL), jnp.float32),
            *[pltpu.SemaphoreType.DMA for _ in range(2 * NBUF)],
        ],
    )
    def k(table_hbm, idx_hbm, pe_hbm, out_hbm, idx_v, *rest):
        bufs = list(rest[:NBUF])
        pe_v = rest[NBUF]
        gsems = list(rest[NBUF + 1 : NBUF + 1 + NBUF])
        ssems = list(rest[NBUF + 1 + NBUF :])

        wid = lax.axis_index("s") * NC + lax.axis_index("c")
        base = pl.multiple_of(wid * ROWS_PER_W, ROWS_PER_W)
        pltpu.sync_copy(pe_hbm, pe_v)
        pltpu.sync_copy(idx_hbm.at[pl.ds(base, ROWS_PER_W)], idx_v)

        def gather_cp(ci, bi):
            loff = pl.multiple_of(ci * CHUNK, CHUNK)
            return pltpu.make_async_copy(
                table_hbm.at[idx_v.at[pl.ds(loff, CHUNK)]], bufs[bi], gsems[bi]
            )

        def scatter_cp(ci, bi):
            goff = pl.multiple_of(base + ci * CHUNK, CHUNK)
            return pltpu.make_async_copy(
                bufs[bi], out_hbm.at[pl.ds(goff, CHUNK)], ssems[bi]
            )

        def compute(ci, bi):
            buf = bufs[bi]
            t = (base + ci * CHUNK) // BATCH
            pe_vecs = [pe_v[t, pl.ds(j * LANES, LANES)] for j in range(NVEC)]

            def row_body(r, _):
                for j in range(NVEC):
                    sl = pl.ds(j * LANES, LANES)
                    buf[r, sl] = buf[r, sl] * SCALE + pe_vecs[j]
                return 0

            lax.fori_loop(0, CHUNK, row_body, 0)

        for bi in range(NBUF):
            gather_cp(bi, bi).start()

        def group_body(g, _):
            for k_ in range(NBUF):
                ci = g * NBUF + k_
                pci = ci - LAG
                nci = pci + NBUF

                @pl.when((ci >= LAG) & (nci < NCHUNKS))
                def _():
                    rbi = (k_ - LAG) % NBUF
                    scatter_cp(pci, rbi).wait()
                    gather_cp(nci, rbi).start()

                gather_cp(ci, k_).wait()
                compute(ci, k_)
                scatter_cp(ci, k_).start()
            return 0

        lax.fori_loop(0, NGROUPS, group_body, 0)

        for ci in range(NCHUNKS - NBUF, NCHUNKS):
            scatter_cp(ci, ci % NBUF).wait()

    return k(table, idx_flat, pe)


def _masks(tgt32):
    def body(tgt_ref, pad_ref, tri_ref):
        pad_ref[...] = tgt_ref[...] == 0
        r = lax.broadcasted_iota(jnp.int32, (SEQ, SEQ), 0)
        c = lax.broadcasted_iota(jnp.int32, (SEQ, SEQ), 1)
        tri_ref[...] = jnp.where(c <= r, 0.0, -jnp.inf).astype(jnp.float32)

    return pl.pallas_call(
        body,
        out_shape=(
            jax.ShapeDtypeStruct((BATCH, SEQ), jnp.bool_),
            jax.ShapeDtypeStruct((SEQ, SEQ), jnp.float32),
        ),
    )(tgt32)


def kernel(tgt, table):
    tgt32 = tgt.astype(jnp.int32)
    idx_flat = jnp.transpose(tgt32).reshape(B)
    emb_flat = _sc_embed(table, idx_flat, jnp.asarray(_PE))
    pad, tri = _masks(tgt32)
    return emb_flat.reshape(SEQ, BATCH, D_MODEL), tri, pad


# speedup vs baseline: 1.1239x; 1.0419x over previous
import functools
import math

import numpy as np
import jax
import jax.numpy as jnp
from jax import lax
from jax.experimental import pallas as pl
from jax.experimental.pallas import tpu as pltpu
from jax.experimental.pallas import tpu_sc as plsc

D_MODEL = 256
BATCH = 1024
SEQ = 50
B = SEQ * BATCH
NC, NS = 2, 16
NW = NC * NS
ROWS_PER_W = B // NW
CHUNK = 64
NCHUNKS = ROWS_PER_W // CHUNK
LANES = 16
NVEC = D_MODEL // LANES
SCALE = 16.0


def _pe_rows():
    position = np.arange(SEQ, dtype=np.float32)[:, None]
    div_term = np.exp(
        np.arange(0, D_MODEL, 2, dtype=np.float32) * -(math.log(10000.0) / D_MODEL)
    )
    pe = np.zeros((SEQ, D_MODEL), dtype=np.float32)
    pe[:, 0::2] = np.sin(position * div_term)
    pe[:, 1::2] = np.cos(position * div_term)
    return pe


_PE = np.pad(_pe_rows(), ((0, 6), (0, 0)))


NBUF = 5
LAG = 2
NGROUPS = NCHUNKS // NBUF


def _sc_embed(table, idx_flat, pe):
    mesh = plsc.VectorSubcoreMesh(core_axis_name="c", subcore_axis_name="s")

    @functools.partial(
        pl.kernel,
        mesh=mesh,
        out_type=jax.ShapeDtypeStruct((B, D_MODEL), jnp.float32),
        scratch_types=[
            pltpu.VMEM((ROWS_PER_W,), jnp.int32),
            *[pltpu.VMEM((CHUNK, D_MODEL), jnp.float32) for _ in range(NBUF)],
            pltpu.VMEM((16, D_MODEL), jnp.float32),
            *[pltpu.SemaphoreType.DMA for _ in range(2 * NBUF)],
        ],
    )
    def k(table_hbm, idx_hbm, pe_hbm, out_hbm, idx_v, *rest):
        bufs = list(rest[:NBUF])
        pe_v = rest[NBUF]
        gsems = list(rest[NBUF + 1 : NBUF + 1 + NBUF])
        ssems = list(rest[NBUF + 1 + NBUF :])

        wid = lax.axis_index("s") * NC + lax.axis_index("c")
        base = pl.multiple_of(wid * ROWS_PER_W, ROWS_PER_W)
        t0 = pl.multiple_of((base // BATCH) // 8 * 8, 8)
        pltpu.sync_copy(pe_hbm.at[pl.ds(t0, 16)], pe_v)
        pltpu.sync_copy(idx_hbm.at[pl.ds(base, ROWS_PER_W)], idx_v)

        def gather_cp(ci, bi):
            loff = pl.multiple_of(ci * CHUNK, CHUNK)
            return pltpu.make_async_copy(
                table_hbm.at[idx_v.at[pl.ds(loff, CHUNK)]], bufs[bi], gsems[bi]
            )

        def scatter_cp(ci, bi):
            goff = pl.multiple_of(base + ci * CHUNK, CHUNK)
            return pltpu.make_async_copy(
                bufs[bi], out_hbm.at[pl.ds(goff, CHUNK)], ssems[bi]
            )

        def compute(ci, bi):
            buf = bufs[bi]
            t = (base + ci * CHUNK) // BATCH - t0
            pe_vecs = [pe_v[t, pl.ds(j * LANES, LANES)] for j in range(NVEC)]

            def row_body(r, _):
                for j in range(NVEC):
                    sl = pl.ds(j * LANES, LANES)
                    buf[r, sl] = buf[r, sl] * SCALE + pe_vecs[j]
                return 0

            lax.fori_loop(0, CHUNK, row_body, 0)

        for bi in range(NBUF):
            gather_cp(bi, bi).start()

        def group_body(g, _):
            for k_ in range(NBUF):
                ci = g * NBUF + k_
                pci = ci - LAG
                nci = pci + NBUF

                @pl.when((ci >= LAG) & (nci < NCHUNKS))
                def _():
                    rbi = (k_ - LAG) % NBUF
                    scatter_cp(pci, rbi).wait()
                    gather_cp(nci, rbi).start()

                gather_cp(ci, k_).wait()
                compute(ci, k_)
                scatter_cp(ci, k_).start()
            return 0

        lax.fori_loop(0, NGROUPS, group_body, 0)

        for ci in range(NCHUNKS - NBUF, NCHUNKS):
            scatter_cp(ci, ci % NBUF).wait()

    return k(table, idx_flat, pe)


def _masks(tgt32):
    def body(tgt_ref, pad_ref, tri_ref):
        pad_ref[...] = tgt_ref[...] == 0
        r = lax.broadcasted_iota(jnp.int32, (SEQ, SEQ), 0)
        c = lax.broadcasted_iota(jnp.int32, (SEQ, SEQ), 1)
        tri_ref[...] = jnp.where(c <= r, 0.0, -jnp.inf).astype(jnp.float32)

    return pl.pallas_call(
        body,
        out_shape=(
            jax.ShapeDtypeStruct((BATCH, SEQ), jnp.bool_),
            jax.ShapeDtypeStruct((SEQ, SEQ), jnp.float32),
        ),
    )(tgt32)


def kernel(tgt, table):
    tgt32 = tgt.astype(jnp.int32)
    idx_flat = jnp.transpose(tgt32).reshape(B)
    emb_flat = _sc_embed(table, idx_flat, jnp.asarray(_PE))
    pad, tri = _masks(tgt32)
    return emb_flat.reshape(SEQ, BATCH, D_MODEL), tri, pad
